# baseline (speedup 1.0000x reference)
import functools

import jax
import jax.numpy as jnp
from jax import lax
from jax.experimental import pallas as pl
from jax.experimental.pallas import tpu as pltpu
from jax.experimental.pallas import tpu_sc as plsc

L = 16
NC = 2
NS = 16
NW = NC * NS
W = 32768
CHUNK = 256


def _transpose_body(dim, tt_ref, out_ref):
    a = tt_ref[...]
    wq = W // 4
    packed = pltpu.bitcast(a.astype(jnp.bfloat16), jnp.int32)
    for p in range(4):
        out_ref[:, (dim // 2) * p:(dim // 2) * (p + 1)] = (
            jnp.transpose(packed[:, p * wq:(p + 1) * wq], (1, 0)))


def _cosine_gather_kernel(dim, bpw, u_idx_hbm, s_idx_hbm, tr_hbm, out_hbm,
                          u_idx_v, s_idx_v, uq_v, sq_v, ucb_v, scb_v,
                          u_buf, s_buf, out_v, sem_u, sem_s):
    wid = lax.axis_index("s") * NC + lax.axis_index("c")
    idx_rows = bpw // 128
    rows_per_chunk = CHUNK // 128
    base = wid * idx_rows

    pltpu.sync_copy(u_idx_hbm.at[pl.ds(base, idx_rows)], u_idx_v)
    pltpu.sync_copy(s_idx_hbm.at[pl.ds(base, idx_rows)], s_idx_v)

    quarter_w = W // 4
    lw = W.bit_length() - 1
    for c in range(idx_rows):
        for l in range(128 // L):
            sl = pl.ds(l * L, L)
            for iv, qv, cbv in ((u_idx_v, uq_v, ucb_v), (s_idx_v, sq_v, scb_v)):
                r = iv[c, sl]
                g_blk = lax.shift_right_logical(r, lw)
                o = jnp.bitwise_and(r, jnp.int32(W - 1))
                row = jnp.bitwise_or(
                    lax.shift_left(g_blk, lw - 2),
                    jnp.bitwise_and(o, jnp.int32(quarter_w - 1)))
                cb = lax.shift_left(
                    lax.shift_right_logical(o, lw - 2), 5)
                qv[c, sl] = row
                cbv[pl.ds(c * 128 + l * L, L)] = cb

    iota16 = lax.iota(jnp.int32, 16)
    n_chunks = idx_rows // rows_per_chunk

    for chunk in range(n_chunks):
        copies = []
        for j in range(rows_per_chunk):
            jj = chunk * rows_per_chunk + j
            copies.append(pltpu.async_copy(
                tr_hbm.at[uq_v.at[jj]], u_buf.at[pl.ds(j * 128, 128)], sem_u))
            copies.append(pltpu.async_copy(
                tr_hbm.at[sq_v.at[jj]], s_buf.at[pl.ds(j * 128, 128)], sem_s))
        for cp in copies:
            cp.wait()

        def group_body(g, carry):
            j_loc = g * L + iota16
            cb_u = ucb_v[pl.ds(chunk * CHUNK + g * L, L)]
            cb_s = scb_v[pl.ds(chunk * CHUNK + g * L, L)]
            num = jnp.zeros((L,), jnp.float32)
            uu = jnp.zeros((L,), jnp.float32)
            ss = jnp.zeros((L,), jnp.float32)
            for k in range(dim // 2):
                kv = jnp.full((L,), k, jnp.int32)
                ui = plsc.load_gather(u_buf, [j_loc, cb_u + kv])
                si = plsc.load_gather(s_buf, [j_loc, cb_s + kv])
                ua, ub = plsc.unpack(plsc.bitcast(ui, jnp.bfloat16),
                                     format=plsc.PackFormat.INTERLEAVED)
                sa, sb = plsc.unpack(plsc.bitcast(si, jnp.bfloat16),
                                     format=plsc.PackFormat.INTERLEAVED)
                num = num + ua * sa + ub * sb
                uu = uu + ua * ua + ub * ub
                ss = ss + sa * sa + sb * sb
            x = (jnp.maximum(uu, jnp.float32(1e-16))
                 * jnp.maximum(ss, jnp.float32(1e-16)))
            xi = lax.bitcast_convert_type(x, jnp.int32)
            yi = jnp.int32(0x5F3759DF) - lax.shift_right_arithmetic(xi, 1)
            y = lax.bitcast_convert_type(yi, jnp.float32)
            half_x = jnp.float32(0.5) * x
            for _ in range(3):
                y = y * (jnp.float32(1.5) - half_x * y * y)
            out_v[pl.ds(g * L, L)] = num * y
            return carry

        lax.fori_loop(0, CHUNK // L, group_body, jnp.int32(0))
        pltpu.sync_copy(
            out_v, out_hbm.at[pl.ds(wid * bpw + chunk * CHUNK, CHUNK)])


def kernel(user_emb, user_emb_offsets, section_emb, section_emb_offsets,
           node2vec_table):
    del user_emb_offsets, section_emb_offsets
    b = user_emb.shape[0]
    vocab, dim = node2vec_table.shape
    bpw = b // NW
    idx_rows = bpw // 128
    grid = (vocab + W - 1) // W
    out_rows = grid * (W // 4)

    tt = jnp.transpose(node2vec_table)
    tr = pl.pallas_call(
        functools.partial(_transpose_body, dim),
        grid=(grid,),
        in_specs=[pl.BlockSpec((dim, W), lambda g: (0, g))],
        out_specs=pl.BlockSpec((W // 4, 2 * dim), lambda g: (g, 0)),
        out_shape=jax.ShapeDtypeStruct((out_rows, 2 * dim), jnp.int32),
        compiler_params=pltpu.CompilerParams(
            vmem_limit_bytes=56 * 1024 * 1024),
    )(tt)

    mesh = plsc.VectorSubcoreMesh(core_axis_name="c", subcore_axis_name="s")
    run = pl.kernel(
        functools.partial(_cosine_gather_kernel, dim, bpw),
        mesh=mesh,
        compiler_params=pltpu.CompilerParams(
            needs_layout_passes=False, use_tc_tiling_on_sc=False),
        out_type=jax.ShapeDtypeStruct((b,), jnp.float32),
        scratch_types=[
            pltpu.VMEM((idx_rows, 128), jnp.int32),
            pltpu.VMEM((idx_rows, 128), jnp.int32),
            pltpu.VMEM((idx_rows, 128), jnp.int32),
            pltpu.VMEM((idx_rows, 128), jnp.int32),
            pltpu.VMEM((idx_rows * 128,), jnp.int32),
            pltpu.VMEM((idx_rows * 128,), jnp.int32),
            pltpu.VMEM((CHUNK, 2 * dim), jnp.int32),
            pltpu.VMEM((CHUNK, 2 * dim), jnp.int32),
            pltpu.VMEM((CHUNK,), jnp.float32),
            pltpu.SemaphoreType.DMA,
            pltpu.SemaphoreType.DMA,
        ],
    )
    u_idx = user_emb.reshape(b // 128, 128).astype(jnp.int32)
    s_idx = section_emb.reshape(b // 128, 128).astype(jnp.int32)
    return run(u_idx, s_idx, tr)


# speedup vs baseline: 1.0684x; 1.0684x over previous
import functools

import jax
import jax.numpy as jnp
from jax import lax
from jax.experimental import pallas as pl
from jax.experimental.pallas import tpu as pltpu
from jax.experimental.pallas import tpu_sc as plsc

L = 16
NC = 2
NS = 16
NW = NC * NS
W = 32768
CHUNK = 256


def _transpose_body(dim, tt_ref, out_ref):
    a = tt_ref[...]
    out_ref[:, 0:dim] = jnp.transpose(a[:, 0:W // 2], (1, 0))
    out_ref[:, dim:2 * dim] = jnp.transpose(a[:, W // 2:W], (1, 0))


def _cosine_gather_kernel(dim, bpw, u_idx_hbm, s_idx_hbm, tr_hbm, out_hbm,
                          u_idx_v, s_idx_v, uq_v, sq_v, ucb_v, scb_v,
                          u_buf, s_buf, out_v, sem_u, sem_s):
    wid = lax.axis_index("s") * NC + lax.axis_index("c")
    idx_rows = bpw // 128
    rows_per_chunk = CHUNK // 128
    base = wid * idx_rows

    pltpu.sync_copy(u_idx_hbm.at[pl.ds(base, idx_rows)], u_idx_v)
    pltpu.sync_copy(s_idx_hbm.at[pl.ds(base, idx_rows)], s_idx_v)

    half_w = W // 2
    lw = W.bit_length() - 1
    for c in range(idx_rows):
        for l in range(128 // L):
            sl = pl.ds(l * L, L)
            for iv, qv, cbv in ((u_idx_v, uq_v, ucb_v), (s_idx_v, sq_v, scb_v)):
                r = iv[c, sl]
                g_blk = lax.shift_right_logical(r, lw)
                o = jnp.bitwise_and(r, jnp.int32(W - 1))
                row = jnp.bitwise_or(
                    lax.shift_left(g_blk, lw - 1),
                    jnp.bitwise_and(o, jnp.int32(half_w - 1)))
                cb = lax.shift_left(
                    lax.shift_right_logical(o, lw - 1), 6)
                qv[c, sl] = row
                cbv[pl.ds(c * 128 + l * L, L)] = cb

    iota16 = lax.iota(jnp.int32, 16)
    n_chunks = idx_rows // rows_per_chunk

    for chunk in range(n_chunks):
        copies = []
        for j in range(rows_per_chunk):
            jj = chunk * rows_per_chunk + j
            copies.append(pltpu.async_copy(
                tr_hbm.at[uq_v.at[jj]], u_buf.at[pl.ds(j * 128, 128)], sem_u))
            copies.append(pltpu.async_copy(
                tr_hbm.at[sq_v.at[jj]], s_buf.at[pl.ds(j * 128, 128)], sem_s))
        for cp in copies:
            cp.wait()

        def group_body(g, carry):
            j_loc = g * L + iota16
            cb_u = ucb_v[pl.ds(chunk * CHUNK + g * L, L)]
            cb_s = scb_v[pl.ds(chunk * CHUNK + g * L, L)]
            num = jnp.zeros((L,), jnp.float32)
            uu = jnp.zeros((L,), jnp.float32)
            ss = jnp.zeros((L,), jnp.float32)
            for k in range(dim):
                kv = jnp.full((L,), k, jnp.int32)
                u = plsc.load_gather(u_buf, [j_loc, cb_u + kv])
                s = plsc.load_gather(s_buf, [j_loc, cb_s + kv])
                num = num + u * s
                uu = uu + u * u
                ss = ss + s * s
            x = (jnp.maximum(uu, jnp.float32(1e-16))
                 * jnp.maximum(ss, jnp.float32(1e-16)))
            xi = lax.bitcast_convert_type(x, jnp.int32)
            yi = jnp.int32(0x5F3759DF) - lax.shift_right_arithmetic(xi, 1)
            y = lax.bitcast_convert_type(yi, jnp.float32)
            half_x = jnp.float32(0.5) * x
            for _ in range(3):
                y = y * (jnp.float32(1.5) - half_x * y * y)
            out_v[pl.ds(g * L, L)] = num * y
            return carry

        lax.fori_loop(0, CHUNK // L, group_body, jnp.int32(0))
        pltpu.sync_copy(
            out_v, out_hbm.at[pl.ds(wid * bpw + chunk * CHUNK, CHUNK)])


def kernel(user_emb, user_emb_offsets, section_emb, section_emb_offsets,
           node2vec_table):
    del user_emb_offsets, section_emb_offsets
    b = user_emb.shape[0]
    vocab, dim = node2vec_table.shape
    bpw = b // NW
    idx_rows = bpw // 128
    grid = (vocab + W - 1) // W
    out_rows = grid * (W // 2)

    tt = jnp.transpose(node2vec_table)
    tr = pl.pallas_call(
        functools.partial(_transpose_body, dim),
        grid=(grid,),
        in_specs=[pl.BlockSpec((dim, W), lambda g: (0, g))],
        out_specs=pl.BlockSpec((W // 2, 2 * dim), lambda g: (g, 0)),
        out_shape=jax.ShapeDtypeStruct((out_rows, 2 * dim), jnp.float32),
    )(tt)

    mesh = plsc.VectorSubcoreMesh(core_axis_name="c", subcore_axis_name="s")
    run = pl.kernel(
        functools.partial(_cosine_gather_kernel, dim, bpw),
        mesh=mesh,
        compiler_params=pltpu.CompilerParams(
            needs_layout_passes=False, use_tc_tiling_on_sc=False),
        out_type=jax.ShapeDtypeStruct((b,), jnp.float32),
        scratch_types=[
            pltpu.VMEM((idx_rows, 128), jnp.int32),
            pltpu.VMEM((idx_rows, 128), jnp.int32),
            pltpu.VMEM((idx_rows, 128), jnp.int32),
            pltpu.VMEM((idx_rows, 128), jnp.int32),
            pltpu.VMEM((idx_rows * 128,), jnp.int32),
            pltpu.VMEM((idx_rows * 128,), jnp.int32),
            pltpu.VMEM((CHUNK, 2 * dim), jnp.float32),
            pltpu.VMEM((CHUNK, 2 * dim), jnp.float32),
            pltpu.VMEM((CHUNK,), jnp.float32),
            pltpu.SemaphoreType.DMA,
            pltpu.SemaphoreType.DMA,
        ],
    )
    u_idx = user_emb.reshape(b // 128, 128).astype(jnp.int32)
    s_idx = section_emb.reshape(b // 128, 128).astype(jnp.int32)
    return run(u_idx, s_idx, tr)


# speedup vs baseline: 1.0860x; 1.0164x over previous
import functools

import jax
import jax.numpy as jnp
from jax import lax
from jax.experimental import pallas as pl
from jax.experimental.pallas import tpu as pltpu
from jax.experimental.pallas import tpu_sc as plsc

L = 16
NC = 2
NS = 16
NW = NC * NS
W = 32768
CHUNK = 256


def _transpose_body(dim, tt_ref, out_ref):
    a = tt_ref[...]
    out_ref[:, 0:dim] = jnp.transpose(a[:, 0:W // 2], (1, 0))
    out_ref[:, dim:2 * dim] = jnp.transpose(a[:, W // 2:W], (1, 0))


def _cosine_gather_kernel(dim, bpw, u_idx_hbm, s_idx_hbm, tr_hbm, out_hbm,
                          u_idx_v, s_idx_v, uq_v, sq_v, ucb_v, scb_v,
                          u_buf, s_buf, out_v,
                          sem_u0, sem_u1, sem_s0, sem_s1):
    wid = lax.axis_index("s") * NC + lax.axis_index("c")
    idx_rows = bpw // 128
    sem_u = (sem_u0, sem_u1)
    sem_s = (sem_s0, sem_s1)
    base = wid * idx_rows

    pltpu.sync_copy(u_idx_hbm.at[pl.ds(base, idx_rows)], u_idx_v)
    pltpu.sync_copy(s_idx_hbm.at[pl.ds(base, idx_rows)], s_idx_v)

    half_w = W // 2
    lw = W.bit_length() - 1
    for c in range(idx_rows):
        for l in range(128 // L):
            sl = pl.ds(l * L, L)
            for iv, qv, cbv in ((u_idx_v, uq_v, ucb_v), (s_idx_v, sq_v, scb_v)):
                r = iv[c, sl]
                g_blk = lax.shift_right_logical(r, lw)
                o = jnp.bitwise_and(r, jnp.int32(W - 1))
                row = jnp.bitwise_or(
                    lax.shift_left(g_blk, lw - 1),
                    jnp.bitwise_and(o, jnp.int32(half_w - 1)))
                cb = lax.shift_left(
                    lax.shift_right_logical(o, lw - 1), 6)
                qv[c, sl] = row
                cbv[pl.ds(c * 128 + l * L, L)] = cb

    iota16 = lax.iota(jnp.int32, 16)

    def sub_copies(c, bb):
        return (pltpu.make_async_copy(
                    tr_hbm.at[uq_v.at[c]], u_buf.at[bb], sem_u[bb]),
                pltpu.make_async_copy(
                    tr_hbm.at[sq_v.at[c]], s_buf.at[bb], sem_s[bb]))

    for cp in sub_copies(0, 0):
        cp.start()

    for chunk in range(idx_rows):
        bb = chunk % 2
        for cp in sub_copies(chunk, bb):
            cp.wait()
        if chunk + 1 < idx_rows:
            for cp in sub_copies(chunk + 1, 1 - bb):
                cp.start()

        def group_body(g, carry, chunk=chunk, bb=bb):
            j_loc = g * L + iota16
            cb_u = ucb_v[pl.ds(chunk * 128 + g * L, L)]
            cb_s = scb_v[pl.ds(chunk * 128 + g * L, L)]
            num = jnp.zeros((L,), jnp.float32)
            uu = jnp.zeros((L,), jnp.float32)
            ss = jnp.zeros((L,), jnp.float32)
            for k in range(dim):
                kv = jnp.full((L,), k, jnp.int32)
                u = plsc.load_gather(u_buf.at[bb], [j_loc, cb_u + kv])
                s = plsc.load_gather(s_buf.at[bb], [j_loc, cb_s + kv])
                num = num + u * s
                uu = uu + u * u
                ss = ss + s * s
            x = (jnp.maximum(uu, jnp.float32(1e-16))
                 * jnp.maximum(ss, jnp.float32(1e-16)))
            xi = lax.bitcast_convert_type(x, jnp.int32)
            yi = jnp.int32(0x5F3759DF) - lax.shift_right_arithmetic(xi, 1)
            y = lax.bitcast_convert_type(yi, jnp.float32)
            half_x = jnp.float32(0.5) * x
            for _ in range(3):
                y = y * (jnp.float32(1.5) - half_x * y * y)
            out_v[pl.ds(chunk * 128 + g * L, L)] = num * y
            return carry

        lax.fori_loop(0, 128 // L, group_body, jnp.int32(0))

    pltpu.sync_copy(out_v, out_hbm.at[pl.ds(wid * bpw, bpw)])


def kernel(user_emb, user_emb_offsets, section_emb, section_emb_offsets,
           node2vec_table):
    del user_emb_offsets, section_emb_offsets
    b = user_emb.shape[0]
    vocab, dim = node2vec_table.shape
    bpw = b // NW
    idx_rows = bpw // 128
    grid = (vocab + W - 1) // W
    out_rows = grid * (W // 2)

    tt = jnp.transpose(node2vec_table)
    tr = pl.pallas_call(
        functools.partial(_transpose_body, dim),
        grid=(grid,),
        in_specs=[pl.BlockSpec((dim, W), lambda g: (0, g))],
        out_specs=pl.BlockSpec((W // 2, 2 * dim), lambda g: (g, 0)),
        out_shape=jax.ShapeDtypeStruct((out_rows, 2 * dim), jnp.float32),
        compiler_params=pltpu.CompilerParams(
            vmem_limit_bytes=56 * 1024 * 1024),
    )(tt)

    mesh = plsc.VectorSubcoreMesh(core_axis_name="c", subcore_axis_name="s")
    run = pl.kernel(
        functools.partial(_cosine_gather_kernel, dim, bpw),
        mesh=mesh,
        compiler_params=pltpu.CompilerParams(
            needs_layout_passes=False, use_tc_tiling_on_sc=False),
        out_type=jax.ShapeDtypeStruct((b,), jnp.float32),
        scratch_types=[
            pltpu.VMEM((idx_rows, 128), jnp.int32),
            pltpu.VMEM((idx_rows, 128), jnp.int32),
            pltpu.VMEM((idx_rows, 128), jnp.int32),
            pltpu.VMEM((idx_rows, 128), jnp.int32),
            pltpu.VMEM((idx_rows * 128,), jnp.int32),
            pltpu.VMEM((idx_rows * 128,), jnp.int32),
            pltpu.VMEM((2, 128, 2 * dim), jnp.float32),
            pltpu.VMEM((2, 128, 2 * dim), jnp.float32),
            pltpu.VMEM((bpw,), jnp.float32),
            pltpu.SemaphoreType.DMA,
            pltpu.SemaphoreType.DMA,
            pltpu.SemaphoreType.DMA,
            pltpu.SemaphoreType.DMA,
        ],
    )
    u_idx = user_emb.reshape(b // 128, 128).astype(jnp.int32)
    s_idx = section_emb.reshape(b // 128, 128).astype(jnp.int32)
    return run(u_idx, s_idx, tr)


# speedup vs baseline: 1.0872x; 1.0011x over previous
import functools

import jax
import jax.numpy as jnp
from jax import lax
from jax.experimental import pallas as pl
from jax.experimental.pallas import tpu as pltpu
from jax.experimental.pallas import tpu_sc as plsc

L = 16
NC = 2
NS = 16
NW = NC * NS
W = 32768


def _transpose_body(dim, tt_ref, out_ref):
    a = tt_ref[...]
    out_ref[:, 0:dim] = jnp.transpose(a[:, 0:W // 2], (1, 0))
    out_ref[:, dim:2 * dim] = jnp.transpose(a[:, W // 2:W], (1, 0))


def _cosine_gather_kernel(dim, bpw, u_idx_hbm, s_idx_hbm, tr_hbm, out_hbm,
                          u_idx_v, s_idx_v, uq_v, sq_v, ucb_v, scb_v,
                          u_buf, s_buf, out_v,
                          sem_u0, sem_u1, sem_s0, sem_s1):
    wid = lax.axis_index("s") * NC + lax.axis_index("c")
    idx_rows = bpw // 128
    sem_u = (sem_u0, sem_u1)
    sem_s = (sem_s0, sem_s1)
    base = wid * idx_rows

    pltpu.sync_copy(u_idx_hbm.at[pl.ds(base, idx_rows)], u_idx_v)
    pltpu.sync_copy(s_idx_hbm.at[pl.ds(base, idx_rows)], s_idx_v)

    half_w = W // 2
    lw = W.bit_length() - 1
    for c in range(idx_rows):
        for l in range(128 // L):
            sl = pl.ds(l * L, L)
            for iv, qv, cbv in ((u_idx_v, uq_v, ucb_v), (s_idx_v, sq_v, scb_v)):
                r = iv[c, sl]
                g_blk = lax.shift_right_logical(r, lw)
                o = jnp.bitwise_and(r, jnp.int32(W - 1))
                row = jnp.bitwise_or(
                    lax.shift_left(g_blk, lw - 1),
                    jnp.bitwise_and(o, jnp.int32(half_w - 1)))
                cb = lax.shift_left(
                    lax.shift_right_logical(o, lw - 1), 6)
                qv[c, sl] = row
                cbv[pl.ds(c * 128 + l * L, L)] = cb

    iota16 = lax.iota(jnp.int32, 16)

    def sub_copies(c, bb):
        return (pltpu.make_async_copy(
                    tr_hbm.at[uq_v.at[c]], u_buf.at[bb], sem_u[bb]),
                pltpu.make_async_copy(
                    tr_hbm.at[sq_v.at[c]], s_buf.at[bb], sem_s[bb]))

    for cp in sub_copies(0, 0):
        cp.start()

    for chunk in range(idx_rows):
        bb = chunk % 2
        for cp in sub_copies(chunk, bb):
            cp.wait()
        if chunk + 1 < idx_rows:
            for cp in sub_copies(chunk + 1, 1 - bb):
                cp.start()

        def group_body(g, carry, chunk=chunk, bb=bb):
            j_loc = g * L + iota16
            cb_u = ucb_v[pl.ds(chunk * 128 + g * L, L)]
            cb_s = scb_v[pl.ds(chunk * 128 + g * L, L)]
            num = jnp.zeros((L,), jnp.float32)
            uu = jnp.zeros((L,), jnp.float32)
            ss = jnp.zeros((L,), jnp.float32)
            for k in range(dim):
                kv = jnp.full((L,), k, jnp.int32)
                u = plsc.load_gather(u_buf.at[bb], [j_loc, cb_u + kv])
                s = plsc.load_gather(s_buf.at[bb], [j_loc, cb_s + kv])
                num = num + u * s
                uu = uu + u * u
                ss = ss + s * s
            x = (jnp.maximum(uu, jnp.float32(1e-16))
                 * jnp.maximum(ss, jnp.float32(1e-16)))
            xi = lax.bitcast_convert_type(x, jnp.int32)
            yi = jnp.int32(0x5F3759DF) - lax.shift_right_arithmetic(xi, 1)
            y = lax.bitcast_convert_type(yi, jnp.float32)
            half_x = jnp.float32(0.5) * x
            for _ in range(3):
                y = y * (jnp.float32(1.5) - half_x * y * y)
            out_v[pl.ds(chunk * 128 + g * L, L)] = num * y
            return carry

        lax.fori_loop(0, 128 // L, group_body, jnp.int32(0))

    pltpu.sync_copy(out_v, out_hbm.at[pl.ds(wid * bpw, bpw)])


def kernel(user_emb, user_emb_offsets, section_emb, section_emb_offsets,
           node2vec_table):
    del user_emb_offsets, section_emb_offsets
    b = user_emb.shape[0]
    vocab, dim = node2vec_table.shape
    bpw = b // NW
    idx_rows = bpw // 128
    grid = (vocab + W - 1) // W
    out_rows = grid * (W // 2)

    tt = jnp.transpose(node2vec_table)
    tr = pl.pallas_call(
        functools.partial(_transpose_body, dim),
        grid=(grid,),
        in_specs=[pl.BlockSpec((dim, W), lambda g: (0, g))],
        out_specs=pl.BlockSpec((W // 2, 2 * dim), lambda g: (g, 0)),
        out_shape=jax.ShapeDtypeStruct((out_rows, 2 * dim), jnp.float32),
        compiler_params=pltpu.CompilerParams(
            vmem_limit_bytes=56 * 1024 * 1024),
    )(tt)

    mesh = plsc.VectorSubcoreMesh(core_axis_name="c", subcore_axis_name="s")
    run = pl.kernel(
        functools.partial(_cosine_gather_kernel, dim, bpw),
        mesh=mesh,
        compiler_params=pltpu.CompilerParams(
            needs_layout_passes=False, use_tc_tiling_on_sc=False),
        out_type=jax.ShapeDtypeStruct((b,), jnp.float32),
        scratch_types=[
            pltpu.VMEM((idx_rows, 128), jnp.int32),
            pltpu.VMEM((idx_rows, 128), jnp.int32),
            pltpu.VMEM((idx_rows, 128), jnp.int32),
            pltpu.VMEM((idx_rows, 128), jnp.int32),
            pltpu.VMEM((idx_rows * 128,), jnp.int32),
            pltpu.VMEM((idx_rows * 128,), jnp.int32),
            pltpu.VMEM((2, 128, 2 * dim), jnp.float32),
            pltpu.VMEM((2, 128, 2 * dim), jnp.float32),
            pltpu.VMEM((bpw,), jnp.float32),
            pltpu.SemaphoreType.DMA,
            pltpu.SemaphoreType.DMA,
            pltpu.SemaphoreType.DMA,
            pltpu.SemaphoreType.DMA,
        ],
    )
    u_idx = user_emb.reshape(b // 128, 128).astype(jnp.int32)
    s_idx = section_emb.reshape(b // 128, 128).astype(jnp.int32)
    return run(u_idx, s_idx, tr)
